# baseline (speedup 1.0000x reference)
import jax
import jax.numpy as jnp
from jax import lax
from jax.experimental import pallas as pl
from jax.experimental.pallas import tpu as pltpu
from jax.experimental.pallas import tpu_sc as plsc

G = 2
N = 10000
D = 128
E = 320000
NC = 2
NS = 16
NW = NC * NS
S_ROWS = 320
NP = NW * S_ROWS
ACC_ROWS = 328
DUMP_ROW = 320
C = 128
CPS = 16
SCE = C * CPS
NBUF = 2
P = 2 * G
LANES = 16
BW = 48
EP = 158 * SCE
ER = EP // C


def _read_scalar(vref, j):
    v = vref[pl.ds(j, LANES)]
    return v[0]


def _make_layer(use_x2):
    mesh = plsc.VectorSubcoreMesh(core_axis_name="c", subcore_axis_name="s")
    scratch = [
        pltpu.VMEM((BW,), jnp.int32),
        pltpu.VMEM((CPS, C), jnp.int32),
        pltpu.VMEM((CPS, C), jnp.int32),
        pltpu.VMEM((NBUF, C, D), jnp.float32),
        pltpu.VMEM((128, D), jnp.float32),
        pltpu.VMEM((128, D), jnp.float32),
        pltpu.VMEM_SHARED((NS * ACC_ROWS, D), jnp.float32),
    ]
    if use_x2:
        scratch.append(pltpu.VMEM((128, D), jnp.float32))
    scratch.extend([pltpu.SemaphoreType.DMA] * (1 + 2 * NBUF))

    def body(*refs):
        if use_x2:
            (tables, srcs, tgts, bounds, x1, x2, out,
             bnd_v, idx_v, tgt_v, rows_v, eacc_v, x1_v, acc_s, x2_v,
             sem, *bsems) = refs
        else:
            (tables, srcs, tgts, bounds, x1, out,
             bnd_v, idx_v, tgt_v, rows_v, eacc_v, x1_v, acc_s,
             sem, *bsems) = refs
            x2 = x2_v = None
        gsem = bsems[:NBUF]
        ssem = bsems[NBUF:]
        sid = lax.axis_index("s")
        wid = sid * NC + lax.axis_index("c")
        row_lo = wid * S_ROWS
        acc_base = sid * ACC_ROWS
        zeros = jnp.zeros((LANES,), jnp.float32)

        for p in range(P):
            pltpu.sync_copy(bounds.at[pl.ds(p * BW, BW)], bnd_v)
            e_lo = _read_scalar(bnd_v, wid)
            e_hi = _read_scalar(bnd_v, wid + 1)
            e0 = (e_lo // SCE) * SCE
            nsc = (e_hi - e0 + (SCE - 1)) // SCE

            def zero_body(r, _):
                for cb in range(D // LANES):
                    eacc_v[r, pl.ds(cb * LANES, LANES)] = zeros
                return 0
            lax.fori_loop(0, 128, zero_body, 0)
            for zb, zn in ((0, 128), (1, 128), (2, ACC_ROWS - 256)):
                pltpu.sync_copy(
                    eacc_v.at[pl.ds(0, zn)],
                    acc_s.at[pl.ds(acc_base + zb * 128, zn)])

            def sc_loop(si, _, p=p, e0=e0, e_lo=e_lo, e_hi=e_hi):
                base = e0 + si * SCE
                erow = pl.multiple_of(p * ER + base // C, 8)
                gi = pltpu.async_copy(
                    srcs.at[pl.ds(erow, CPS)], idx_v, sem)
                gt = pltpu.async_copy(
                    tgts.at[pl.ds(erow, CPS)], tgt_v, gsem[0])
                gi.wait()
                gt.wait()
                for b in range(SCE // LANES):
                    jj, bb = divmod(b, C // LANES)
                    sl = pl.ds(bb * LANES, LANES)
                    eidx = base + b * LANES + lax.iota(jnp.int32, LANES)
                    ok = (eidx >= e_lo) & (eidx < e_hi)
                    tgt_v[jj, sl] = acc_base + jnp.where(
                        ok, tgt_v[jj, sl] - row_lo, jnp.int32(DUMP_ROW))
                    idx_v[jj, sl] = idx_v[jj, sl] + jnp.int32(p * NP)
                gd = {}
                sd = {}
                for j in range(CPS):
                    bj = j % NBUF
                    if j >= NBUF:
                        sd[j - NBUF].wait()
                    gd[j] = pltpu.async_copy(
                        tables.at[idx_v.at[j]], rows_v.at[bj], gsem[bj])
                    if j > 0:
                        gd[j - 1].wait()
                        sd[j - 1] = pltpu.async_copy(
                            rows_v.at[(j - 1) % NBUF],
                            acc_s.at[tgt_v.at[j - 1]],
                            ssem[(j - 1) % NBUF], add=True)
                gd[CPS - 1].wait()
                sd[CPS - 1] = pltpu.async_copy(
                    rows_v.at[(CPS - 1) % NBUF],
                    acc_s.at[tgt_v.at[CPS - 1]],
                    ssem[(CPS - 1) % NBUF], add=True)
                for j in range(CPS - NBUF, CPS):
                    sd[j].wait()
                return 0
            lax.fori_loop(0, nsc, sc_loop, 0)

            for rb in range(3):
                nr = min(128, S_ROWS - rb * 128)
                r0 = row_lo + rb * 128
                pltpu.sync_copy(acc_s.at[pl.ds(acc_base + rb * 128, nr)],
                                eacc_v.at[pl.ds(0, nr)])
                pltpu.sync_copy(x1.at[p, pl.ds(r0, nr)], x1_v.at[pl.ds(0, nr)])
                if use_x2:
                    pltpu.sync_copy(x2.at[p, pl.ds(r0, nr)],
                                    x2_v.at[pl.ds(0, nr)])

                def ew_body(r, _):
                    for cb in range(D // LANES):
                        sl = pl.ds(cb * LANES, LANES)
                        a = eacc_v[r, sl]
                        y = jnp.maximum(a, 0.01 * a) + x1_v[r, sl]
                        if use_x2:
                            y = y + 2.0 * x2_v[r, sl]
                        rows_v[0, r, sl] = y
                    return 0
                lax.fori_loop(0, nr, ew_body, 0)
                pltpu.sync_copy(rows_v.at[0, pl.ds(0, nr)],
                                out.at[p, pl.ds(r0, nr)])

    return pl.kernel(
        body,
        out_type=jax.ShapeDtypeStruct((P, NP, D), jnp.float32),
        mesh=mesh,
        scratch_types=scratch,
    )


_layer1 = _make_layer(use_x2=False)
_layer2 = _make_layer(use_x2=True)


def kernel(user_embeddings, item_embeddings, adj_src, adj_tgt, tp_src, tp_tgt):
    u0 = jnp.pad(user_embeddings, ((0, 0), (0, NP - N), (0, 0)))
    i0 = jnp.pad(item_embeddings, ((0, 0), (0, NP - N), (0, 0)))
    srcs = jnp.stack([adj_src[0], tp_src[0], adj_src[1], tp_src[1]])
    tgts = jnp.stack([adj_tgt[0], tp_tgt[0], adj_tgt[1], tp_tgt[1]])
    srcs = srcs.astype(jnp.int32)
    tgts = tgts.astype(jnp.int32)
    row_bnds = jnp.arange(NW + 1, dtype=jnp.int32) * S_ROWS
    bounds = jax.vmap(
        lambda t: jnp.searchsorted(t, row_bnds, side="left"))(tgts)
    bounds = jnp.pad(bounds.astype(jnp.int32),
                     ((0, 0), (0, BW - (NW + 1)))).reshape(P * BW)
    srcs_p = jnp.pad(srcs, ((0, 0), (0, EP - E))).reshape(P * ER, C)
    tgts_p = jnp.pad(tgts, ((0, 0), (0, EP - E))).reshape(P * ER, C)

    x1 = jnp.stack([u0[0], i0[0], u0[1], i0[1]])
    tab1 = jnp.stack([i0[0], u0[0], i0[1], u0[1]]).reshape(P * NP, D)
    out1 = _layer1(tab1, srcs_p, tgts_p, bounds, x1)
    tab2 = jnp.stack([out1[1], out1[0], out1[3], out1[2]]).reshape(P * NP, D)
    out2 = _layer2(tab2, srcs_p, tgts_p, bounds, x1, out1)

    user_vector = jnp.stack([out2[0, :N], out2[2, :N]])
    item_vector = jnp.stack([out2[1, :N], out2[3, :N]])
    return (user_vector, item_vector)


# speedup vs baseline: 1.0501x; 1.0501x over previous
import jax
import jax.numpy as jnp
from jax import lax
from jax.experimental import pallas as pl
from jax.experimental.pallas import tpu as pltpu
from jax.experimental.pallas import tpu_sc as plsc

G = 2
N = 10000
D = 128
E = 320000
NC = 2
NS = 16
NW = NC * NS
S_ROWS = 320
NP = NW * S_ROWS
ACC_ROWS = 328
DUMP_ROW = 320
C = 128
CPS = 16
SCE = C * CPS
NBUF = 2
P = 2 * G
LANES = 16
BW = 48
EP = 158 * SCE
ER = EP // C


def _read_scalar(vref, j):
    v = vref[pl.ds(j, LANES)]
    return v[0]


def _make_layer(use_x2):
    mesh = plsc.VectorSubcoreMesh(core_axis_name="c", subcore_axis_name="s")
    scratch = [
        pltpu.VMEM((BW,), jnp.int32),
        pltpu.VMEM((CPS, C), jnp.int32),
        pltpu.VMEM((CPS, C), jnp.int32),
        pltpu.VMEM((NBUF, C, D), jnp.float32),
        pltpu.VMEM((128, D), jnp.float32),
        pltpu.VMEM((128, D), jnp.float32),
        pltpu.VMEM_SHARED((NS * ACC_ROWS, D), jnp.float32),
    ]
    if use_x2:
        scratch.append(pltpu.VMEM((128, D), jnp.float32))
    scratch.extend([pltpu.SemaphoreType.DMA] * (1 + 2 * NBUF))

    def body(*refs):
        if use_x2:
            (tables, srcs, tgts, bounds, x1, x2, out,
             bnd_v, idx_v, tgt_v, rows_v, eacc_v, x1_v, acc_s, x2_v,
             sem, *bsems) = refs
        else:
            (tables, srcs, tgts, bounds, x1, out,
             bnd_v, idx_v, tgt_v, rows_v, eacc_v, x1_v, acc_s,
             sem, *bsems) = refs
            x2 = x2_v = None
        gsem = bsems[:NBUF]
        ssem = bsems[NBUF:]
        sid = lax.axis_index("s")
        wid = sid * NC + lax.axis_index("c")
        row_lo = wid * S_ROWS
        acc_base = sid * ACC_ROWS
        zeros = jnp.zeros((LANES,), jnp.float32)

        for p in range(P):
            pltpu.sync_copy(bounds.at[pl.ds(p * BW, BW)], bnd_v)
            e_lo = _read_scalar(bnd_v, wid)
            e_hi = _read_scalar(bnd_v, wid + 1)
            e0 = (e_lo // 1024) * 1024
            nsc = (e_hi - e0 + (SCE - 1)) // SCE

            def zero_body(r, _):
                for cb in range(D // LANES):
                    eacc_v[r, pl.ds(cb * LANES, LANES)] = zeros
                return 0
            lax.fori_loop(0, 128, zero_body, 0)
            for zb, zn in ((0, 128), (1, 128), (2, ACC_ROWS - 256)):
                pltpu.sync_copy(
                    eacc_v.at[pl.ds(0, zn)],
                    acc_s.at[pl.ds(acc_base + zb * 128, zn)])

            def sc_loop(si, _, p=p, e0=e0, e_lo=e_lo, e_hi=e_hi):
                base = e0 + si * SCE
                erow = pl.multiple_of(p * ER + base // C, 8)
                gi = pltpu.async_copy(
                    srcs.at[pl.ds(erow, CPS)], idx_v, sem)
                gt = pltpu.async_copy(
                    tgts.at[pl.ds(erow, CPS)], tgt_v, gsem[0])
                gi.wait()
                gt.wait()
                for b in range(SCE // LANES):
                    jj, bb = divmod(b, C // LANES)
                    sl = pl.ds(bb * LANES, LANES)
                    eidx = base + b * LANES + lax.iota(jnp.int32, LANES)
                    ok = (eidx >= e_lo) & (eidx < e_hi)
                    tgt_v[jj, sl] = acc_base + jnp.where(
                        ok, tgt_v[jj, sl] - row_lo, jnp.int32(DUMP_ROW))
                    idx_v[jj, sl] = idx_v[jj, sl] + jnp.int32(p * NP)
                gd = {}
                sd = {}
                for j in range(CPS):
                    bj = j % NBUF
                    if j >= NBUF:
                        sd[j - NBUF].wait()
                    gd[j] = pltpu.async_copy(
                        tables.at[idx_v.at[j]], rows_v.at[bj], gsem[bj])
                    if j > 0:
                        gd[j - 1].wait()
                        sd[j - 1] = pltpu.async_copy(
                            rows_v.at[(j - 1) % NBUF],
                            acc_s.at[tgt_v.at[j - 1]],
                            ssem[(j - 1) % NBUF], add=True)
                gd[CPS - 1].wait()
                sd[CPS - 1] = pltpu.async_copy(
                    rows_v.at[(CPS - 1) % NBUF],
                    acc_s.at[tgt_v.at[CPS - 1]],
                    ssem[(CPS - 1) % NBUF], add=True)
                for j in range(CPS - NBUF, CPS):
                    sd[j].wait()
                return 0
            lax.fori_loop(0, nsc, sc_loop, 0)

            for rb in range(3):
                nr = min(128, S_ROWS - rb * 128)
                r0 = row_lo + rb * 128
                pltpu.sync_copy(acc_s.at[pl.ds(acc_base + rb * 128, nr)],
                                eacc_v.at[pl.ds(0, nr)])
                pltpu.sync_copy(x1.at[p, pl.ds(r0, nr)], x1_v.at[pl.ds(0, nr)])
                if use_x2:
                    pltpu.sync_copy(x2.at[p, pl.ds(r0, nr)],
                                    x2_v.at[pl.ds(0, nr)])

                def ew_body(r, _):
                    for cb in range(D // LANES):
                        sl = pl.ds(cb * LANES, LANES)
                        a = eacc_v[r, sl]
                        y = jnp.maximum(a, 0.01 * a) + x1_v[r, sl]
                        if use_x2:
                            y = y + 2.0 * x2_v[r, sl]
                        rows_v[0, r, sl] = y
                    return 0
                lax.fori_loop(0, nr, ew_body, 0)
                pltpu.sync_copy(rows_v.at[0, pl.ds(0, nr)],
                                out.at[p, pl.ds(r0, nr)])

    return pl.kernel(
        body,
        out_type=jax.ShapeDtypeStruct((P, NP, D), jnp.float32),
        mesh=mesh,
        scratch_types=scratch,
    )


_layer1 = _make_layer(use_x2=False)
_layer2 = _make_layer(use_x2=True)


def kernel(user_embeddings, item_embeddings, adj_src, adj_tgt, tp_src, tp_tgt):
    u0 = jnp.pad(user_embeddings, ((0, 0), (0, NP - N), (0, 0)))
    i0 = jnp.pad(item_embeddings, ((0, 0), (0, NP - N), (0, 0)))
    srcs = jnp.stack([adj_src[0], tp_src[0], adj_src[1], tp_src[1]])
    tgts = jnp.stack([adj_tgt[0], tp_tgt[0], adj_tgt[1], tp_tgt[1]])
    srcs = srcs.astype(jnp.int32)
    tgts = tgts.astype(jnp.int32)
    row_bnds = jnp.arange(NW + 1, dtype=jnp.int32) * S_ROWS
    bounds = jax.vmap(
        lambda t: jnp.searchsorted(t, row_bnds, side="left"))(tgts)
    bounds = jnp.pad(bounds.astype(jnp.int32),
                     ((0, 0), (0, BW - (NW + 1)))).reshape(P * BW)
    srcs_p = jnp.pad(srcs, ((0, 0), (0, EP - E))).reshape(P * ER, C)
    tgts_p = jnp.pad(tgts, ((0, 0), (0, EP - E))).reshape(P * ER, C)

    x1 = jnp.stack([u0[0], i0[0], u0[1], i0[1]])
    tab1 = jnp.stack([i0[0], u0[0], i0[1], u0[1]]).reshape(P * NP, D)
    out1 = _layer1(tab1, srcs_p, tgts_p, bounds, x1)
    tab2 = jnp.stack([out1[1], out1[0], out1[3], out1[2]]).reshape(P * NP, D)
    out2 = _layer2(tab2, srcs_p, tgts_p, bounds, x1, out1)

    user_vector = jnp.stack([out2[0, :N], out2[2, :N]])
    item_vector = jnp.stack([out2[1, :N], out2[3, :N]])
    return (user_vector, item_vector)


# speedup vs baseline: 1.3749x; 1.3092x over previous
import jax
import jax.numpy as jnp
from jax import lax
from jax.experimental import pallas as pl
from jax.experimental.pallas import tpu as pltpu
from jax.experimental.pallas import tpu_sc as plsc

G = 2
N = 10000
D = 128
E = 320000
NC = 2
NS = 16
NW = NC * NS
S_ROWS = 320
NP = NW * S_ROWS
ACC_ROWS = 328
DUMP_ROW = 320
C = 128
CPS = 8
SCE = C * CPS
NBUF = 2
P = 2 * G
LANES = 16
BW = 48
EP = 314 * SCE
ER = EP // C


def _read_scalar(vref, j):
    v = vref[pl.ds(j, LANES)]
    return v[0]


def _make_gather_segsum():
    mesh = plsc.VectorSubcoreMesh(core_axis_name="c", subcore_axis_name="s")
    scratch = [
        pltpu.VMEM((BW,), jnp.int32),
        pltpu.VMEM((SCE,), jnp.int32),
        pltpu.VMEM((SCE,), jnp.int32),
        pltpu.VMEM((CPS, C), jnp.int32),
        pltpu.VMEM((NBUF, C, D), jnp.float32),
        pltpu.VMEM((128, D), jnp.float32),
        pltpu.VMEM_SHARED((NS * ACC_ROWS, D), jnp.float32),
    ]
    scratch.extend([pltpu.SemaphoreType.DMA] * (1 + 2 * NBUF))

    def body(tables, srcs, tgts, bounds, out_acc,
             bnd_v, idx_v, tgt_raw, tgt_v, rows_b, zero_v, acc_s,
             sem, *bsems):
        gsem = bsems[:NBUF]
        ssem = bsems[NBUF:]
        sid = lax.axis_index("s")
        wid = sid * NC + lax.axis_index("c")
        row_lo = wid * S_ROWS
        acc_base = sid * ACC_ROWS
        zeros = jnp.zeros((LANES,), jnp.float32)

        def zfill(r, _):
            for cb in range(D // LANES):
                zero_v[r, pl.ds(cb * LANES, LANES)] = zeros
            return 0
        lax.fori_loop(0, 128, zfill, 0)

        for p in range(P):
            pltpu.sync_copy(bounds.at[pl.ds(p * BW, BW)], bnd_v)
            e_lo = _read_scalar(bnd_v, wid)
            e_hi = _read_scalar(bnd_v, wid + 1)
            e0 = (e_lo // 8) * 8
            nsc = (e_hi - e0 + (SCE - 1)) // SCE

            for zb, zn in ((0, 128), (1, 128), (2, ACC_ROWS - 256)):
                pltpu.sync_copy(
                    zero_v.at[pl.ds(0, zn)],
                    acc_s.at[pl.ds(acc_base + zb * 128, zn)])

            def sc_loop(si, _, p=p, e0=e0, e_lo=e_lo, e_hi=e_hi):
                base = e0 + si * SCE
                off = pl.multiple_of(p * EP + base, 8)
                gi = pltpu.async_copy(
                    srcs.at[pl.ds(off, SCE)], idx_v, sem)
                gt = pltpu.async_copy(
                    tgts.at[pl.ds(off, SCE)], tgt_raw, gsem[0])
                gi.wait()
                gt.wait()
                for b in range(SCE // LANES):
                    jj, bb = divmod(b, C // LANES)
                    sl1 = pl.ds(b * LANES, LANES)
                    sl = pl.ds(bb * LANES, LANES)
                    eidx = base + b * LANES + lax.iota(jnp.int32, LANES)
                    ok = (eidx >= e_lo) & (eidx < e_hi)
                    tgt_v[jj, sl] = acc_base + jnp.where(
                        ok, tgt_raw[sl1] - row_lo, jnp.int32(DUMP_ROW))
                    idx_v[sl1] = idx_v[sl1] + jnp.int32(p * NP)
                gd = {}
                sd = {}
                for j in range(CPS):
                    bj = j % NBUF
                    if j >= NBUF:
                        sd[j - NBUF].wait()
                    gd[j] = pltpu.async_copy(
                        tables.at[idx_v.at[pl.ds(j * C, C)]],
                        rows_b.at[bj], gsem[bj])
                    if j > 0:
                        gd[j - 1].wait()
                        sd[j - 1] = pltpu.async_copy(
                            rows_b.at[(j - 1) % NBUF],
                            acc_s.at[tgt_v.at[j - 1]],
                            ssem[(j - 1) % NBUF], add=True)
                gd[CPS - 1].wait()
                sd[CPS - 1] = pltpu.async_copy(
                    rows_b.at[(CPS - 1) % NBUF],
                    acc_s.at[tgt_v.at[CPS - 1]],
                    ssem[(CPS - 1) % NBUF], add=True)
                for j in range(CPS - NBUF, CPS):
                    sd[j].wait()
                return 0
            lax.fori_loop(0, nsc, sc_loop, 0)

            pltpu.sync_copy(acc_s.at[pl.ds(acc_base, S_ROWS)],
                            out_acc.at[p, pl.ds(row_lo, S_ROWS)])

    return pl.kernel(
        body,
        out_type=jax.ShapeDtypeStruct((P, NP, D), jnp.float32),
        mesh=mesh,
        scratch_types=scratch,
    )


_gather_segsum = _make_gather_segsum()

_EB = 1024


def _ep1_body(acc_ref, x1_ref, out_ref):
    a = acc_ref[...]
    out_ref[...] = jnp.maximum(a, 0.01 * a) + x1_ref[...]


def _ep2_body(acc_ref, x1_ref, x2_ref, out_ref):
    a = acc_ref[...]
    out_ref[...] = jnp.maximum(a, 0.01 * a) + x1_ref[...] + 2.0 * x2_ref[...]


def _bs():
    return pl.BlockSpec((1, _EB, D), lambda p, r: (p, r, 0))


_epilogue1 = pl.pallas_call(
    _ep1_body,
    grid=(P, NP // _EB),
    in_specs=[_bs(), _bs()],
    out_specs=_bs(),
    out_shape=jax.ShapeDtypeStruct((P, NP, D), jnp.float32),
)

_epilogue2 = pl.pallas_call(
    _ep2_body,
    grid=(P, NP // _EB),
    in_specs=[_bs(), _bs(), _bs()],
    out_specs=_bs(),
    out_shape=jax.ShapeDtypeStruct((P, NP, D), jnp.float32),
)


def kernel(user_embeddings, item_embeddings, adj_src, adj_tgt, tp_src, tp_tgt):
    u0 = jnp.pad(user_embeddings, ((0, 0), (0, NP - N), (0, 0)))
    i0 = jnp.pad(item_embeddings, ((0, 0), (0, NP - N), (0, 0)))
    srcs = jnp.stack([adj_src[0], tp_src[0], adj_src[1], tp_src[1]])
    tgts = jnp.stack([adj_tgt[0], tp_tgt[0], adj_tgt[1], tp_tgt[1]])
    srcs = srcs.astype(jnp.int32)
    tgts = tgts.astype(jnp.int32)
    row_bnds = jnp.arange(NW + 1, dtype=jnp.int32) * S_ROWS
    bounds = jax.vmap(
        lambda t: jnp.searchsorted(t, row_bnds, side="left"))(tgts)
    bounds = jnp.pad(bounds.astype(jnp.int32),
                     ((0, 0), (0, BW - (NW + 1)))).reshape(P * BW)
    srcs_p = jnp.pad(srcs, ((0, 0), (0, EP - E))).reshape(P * EP)
    tgts_p = jnp.pad(tgts, ((0, 0), (0, EP - E))).reshape(P * EP)

    x1 = jnp.stack([u0[0], i0[0], u0[1], i0[1]])
    tab1 = jnp.stack([i0[0], u0[0], i0[1], u0[1]]).reshape(P * NP, D)
    acc1 = _gather_segsum(tab1, srcs_p, tgts_p, bounds)
    out1 = _epilogue1(acc1, x1)
    tab2 = jnp.stack([out1[1], out1[0], out1[3], out1[2]])
    acc2 = _gather_segsum(tab2.reshape(P * NP, D), srcs_p, tgts_p, bounds)
    out2 = _epilogue2(acc2, x1, out1)

    user_vector = jnp.stack([out2[0, :N], out2[2, :N]])
    item_vector = jnp.stack([out2[1, :N], out2[3, :N]])
    return (user_vector, item_vector)
